# Initial kernel scaffold; baseline (speedup 1.0000x reference)
import jax
import jax.numpy as jnp
from jax.experimental import pallas as pl


def kernel(pred, pred_n, y, cat_candi):
    raise NotImplementedError("write your pallas kernel here")


# speedup vs baseline: 4.2962x; 4.2962x over previous
import functools

import jax
import jax.numpy as jnp
from jax import lax
from jax.experimental import pallas as pl
from jax.experimental.pallas import tpu as pltpu
from jax.experimental.pallas import tpu_sc as plsc

_N = 16384
_C = 64
_NC = 2
_NS = 16
_NW = _NC * _NS
_RPT = _N // _NW
_PRIOR = 0.3
_ALPHA = 1.0

_mesh = plsc.VectorSubcoreMesh(core_axis_name="c", subcore_axis_name="s")


@functools.partial(
    pl.kernel,
    mesh=_mesh,
    out_type=jax.ShapeDtypeStruct((_NW, 4, _C), jnp.float32),
    scratch_types=[
        pltpu.VMEM((128, _C), jnp.float32),
        pltpu.VMEM((128, _C), jnp.int32),
        pltpu.VMEM((128,), jnp.int32),
        pltpu.VMEM((4, _C), jnp.float32),
    ],
)
def _nu_partials(pn_hbm, cc_hbm, y_hbm, out_hbm, pn_v, cc_v, y_v, o_v):
    wid = lax.axis_index("s") * _NC + lax.axis_index("c")
    base = wid * _RPT
    zeros = jnp.zeros((16,), jnp.float32)
    cvecs = [lax.broadcasted_iota(jnp.int32, (16,), 0) + (16 * k) for k in range(4)]

    def grp(g, carry):
        accs = list(carry)
        yv = y_v[pl.ds(g * 16, 16)]
        for j in range(16):
            r = g * 16 + j
            yr = yv[j]
            lab = yr != jnp.int32(-1)
            for k in range(4):
                pn = pn_v[r, pl.ds(16 * k, 16)]
                ccf = cc_v[r, pl.ds(16 * k, 16)].astype(jnp.float32)
                cne = jnp.where(cvecs[k] != yr, 1.0, 0.0).astype(jnp.float32)
                mneg = jnp.where(lab, cne, 1.0 - ccf)
                munl = jnp.where(lab, zeros, ccf)
                accs[4 * k + 0] = accs[4 * k + 0] + mneg * pn
                accs[4 * k + 1] = accs[4 * k + 1] + mneg
                accs[4 * k + 2] = accs[4 * k + 2] + munl * pn
                accs[4 * k + 3] = accs[4 * k + 3] + munl
        return tuple(accs)

    accs = (zeros,) * 16
    for c in range(_RPT // 128):
        pltpu.sync_copy(pn_hbm.at[pl.ds(base + c * 128, 128)], pn_v)
        pltpu.sync_copy(cc_hbm.at[pl.ds(base + c * 128, 128)], cc_v)
        pltpu.sync_copy(y_hbm.at[pl.ds(base + c * 128, 128)], y_v)
        accs = lax.fori_loop(0, 128 // 16, grp, accs)
    for k in range(4):
        o_v[0, pl.ds(16 * k, 16)] = accs[4 * k + 0]
        o_v[1, pl.ds(16 * k, 16)] = accs[4 * k + 1]
        o_v[2, pl.ds(16 * k, 16)] = accs[4 * k + 2]
        o_v[3, pl.ds(16 * k, 16)] = accs[4 * k + 3]
    pltpu.sync_copy(o_v, out_hbm.at[wid])


_BLK = 1024
_GRID = _N // _BLK


def _tc_body(pred_ref, y_ref, part_ref, out_ref, acc):
    i = pl.program_id(0)

    @pl.when(i == 0)
    def _init():
        acc[0] = 0.0
        acc[1] = 0.0

    x = pred_ref[...]
    yb = y_ref[...]
    lab = yb != -1
    m = jnp.max(x, axis=1, keepdims=True)
    e = jnp.exp(x - m)
    s = jnp.sum(e, axis=1, keepdims=True)
    lse = m + jnp.log(s)
    lanes = lax.broadcasted_iota(jnp.int32, x.shape, 1)
    onehot = (lanes == yb) & lab
    picked_sum = jnp.sum(jnp.where(onehot, x, 0.0))
    lse_sum = jnp.sum(jnp.where(lab, lse, 0.0))
    nlab = jnp.sum(jnp.where(lab, 1.0, 0.0))
    acc[0] += picked_sum - lse_sum
    acc[1] += nlab

    @pl.when(i == pl.num_programs(0) - 1)
    def _fin():
        p = jnp.sum(part_ref[...], axis=0)
        s1 = p[0:1, :]
        nn = p[1:2, :]
        sp = p[2:3, :]
        nu = p[3:4, :]
        n_neg = jnp.maximum(nn, 1.0)
        n_unl = jnp.maximum(nu, 1.0)
        neg_risk = _PRIOR * s1 / n_neg
        pos_risk = -_PRIOR * (nn - s1) / n_neg + (nu - sp) / n_unl
        loss = jnp.where(pos_risk < 0, neg_risk, pos_risk + neg_risk)
        nu_total = jnp.sum(loss)
        nl = acc[1]
        ce = jnp.where(nl > 0, -acc[0] / jnp.maximum(nl, 1.0), 0.0)
        out_ref[...] = jnp.full((1, 1), nu_total + _ALPHA * ce, jnp.float32)


def kernel(pred, pred_n, y, cat_candi):
    parts = _nu_partials(pred_n, cat_candi, y)
    out = pl.pallas_call(
        _tc_body,
        grid=(_GRID,),
        in_specs=[
            pl.BlockSpec((_BLK, _C), lambda i: (i, 0)),
            pl.BlockSpec((_BLK, 1), lambda i: (i, 0)),
            pl.BlockSpec((_NW, 4, _C), lambda i: (0, 0, 0)),
        ],
        out_specs=pl.BlockSpec((1, 1), lambda i: (0, 0)),
        out_shape=jax.ShapeDtypeStruct((1, 1), jnp.float32),
        scratch_shapes=[pltpu.SMEM((2,), jnp.float32)],
    )(pred, y.reshape(_N, 1), parts)
    return out[0, 0]
